# initial kernel scaffold (unmeasured)
import jax
import jax.numpy as jnp
from jax import lax
from jax.experimental import pallas as pl
from jax.experimental.pallas import tpu as pltpu

T = 512
D = 1024
V_LOCAL = 8192


def kernel(x, W, labels):
    labels2 = labels.reshape(T, 1)

    def body(x_ref, w_ref, lab_ref, out_ref, stats_ref, peer_ref, send_sem, recv_sem):
        my_x = lax.axis_index("x")
        my_y = lax.axis_index("y")
        my_z = lax.axis_index("z")
        partner = (my_x, my_y, 1 - my_z)

        xb = x_ref[...].astype(jnp.bfloat16)
        wb = w_ref[...].astype(jnp.bfloat16)
        logits = jnp.dot(xb, wb, preferred_element_type=jnp.float32)

        m = jnp.max(logits, axis=1, keepdims=True)
        s = jnp.sum(jnp.exp(logits - m), axis=1, keepdims=True)

        lab_local = lab_ref[...] - my_z * V_LOCAL
        ids = lax.broadcasted_iota(jnp.int32, (T, V_LOCAL), 1)
        lab_logit = jnp.sum(
            jnp.where(ids == lab_local, logits, 0.0), axis=1, keepdims=True
        )

        stats_ref[...] = jnp.concatenate([m, s, lab_logit], axis=1)

        barrier = pltpu.get_barrier_semaphore()
        pl.semaphore_signal(
            barrier, inc=1, device_id=partner, device_id_type=pl.DeviceIdType.MESH
        )
        pl.semaphore_wait(barrier, 1)

        rdma = pltpu.make_async_remote_copy(
            src_ref=stats_ref,
            dst_ref=peer_ref,
            send_sem=send_sem,
            recv_sem=recv_sem,
            device_id=partner,
            device_id_type=pl.DeviceIdType.MESH,
        )
        rdma.start()
        rdma.wait()

        m0 = stats_ref[:, 0:1]
        s0 = stats_ref[:, 1:2]
        l0 = stats_ref[:, 2:3]
        m1 = peer_ref[:, 0:1]
        s1 = peer_ref[:, 1:2]
        l1 = peer_ref[:, 2:3]
        mm = jnp.maximum(m0, m1)
        lse = mm + jnp.log(s0 * jnp.exp(m0 - mm) + s1 * jnp.exp(m1 - mm))
        out_ref[...] = lse - (l0 + l1)

    out = pl.pallas_call(
        body,
        out_shape=jax.ShapeDtypeStruct((T, 1), jnp.float32),
        in_specs=[
            pl.BlockSpec(memory_space=pltpu.VMEM),
            pl.BlockSpec(memory_space=pltpu.VMEM),
            pl.BlockSpec(memory_space=pltpu.VMEM),
        ],
        out_specs=pl.BlockSpec(memory_space=pltpu.VMEM),
        scratch_shapes=[
            pltpu.VMEM((T, 3), jnp.float32),
            pltpu.VMEM((T, 3), jnp.float32),
            pltpu.SemaphoreType.DMA,
            pltpu.SemaphoreType.DMA,
        ],
        compiler_params=pltpu.CompilerParams(collective_id=0),
    )(x, W, labels2)
    return out.reshape(T)


# baseline (device time: 34536 ns/iter reference)
import jax
import jax.numpy as jnp
from jax import lax
from jax.experimental import pallas as pl
from jax.experimental.pallas import tpu as pltpu

T = 512
D = 1024
V_LOCAL = 8192


def kernel(x, W, labels):
    labels2 = labels.reshape(T, 1)

    def body(x_ref, w_ref, lab_ref, out_ref, stats_ref, peer_ref, send_sem, recv_sem):
        my_x = lax.axis_index("x")
        my_y = lax.axis_index("y")
        my_z = lax.axis_index("z")
        partner = (my_x, my_y, 1 - my_z)

        xb = x_ref[...].astype(jnp.bfloat16)
        wb = w_ref[...].astype(jnp.bfloat16)
        logits = jnp.dot(xb, wb, preferred_element_type=jnp.float32)

        m = jnp.max(logits, axis=1, keepdims=True)
        s = jnp.sum(jnp.exp(logits - m), axis=1, keepdims=True)

        lab_local = lab_ref[...] - my_z * V_LOCAL
        ids = lax.broadcasted_iota(jnp.int32, (T, V_LOCAL), 1)
        lab_logit = jnp.sum(
            jnp.where(ids == lab_local, logits, 0.0), axis=1, keepdims=True
        )

        stats_ref[...] = jnp.concatenate([m, s, lab_logit], axis=1)

        barrier = pltpu.get_barrier_semaphore()
        pl.semaphore_signal(
            barrier, inc=1, device_id=partner, device_id_type=pl.DeviceIdType.MESH
        )
        pl.semaphore_wait(barrier, 1)

        rdma = pltpu.make_async_remote_copy(
            src_ref=stats_ref,
            dst_ref=peer_ref,
            send_sem=send_sem,
            recv_sem=recv_sem,
            device_id=partner,
            device_id_type=pl.DeviceIdType.MESH,
        )
        rdma.start()
        rdma.wait()

        m0 = stats_ref[:, 0:1]
        s0 = stats_ref[:, 1:2]
        l0 = stats_ref[:, 2:3]
        m1 = peer_ref[:, 0:1]
        s1 = peer_ref[:, 1:2]
        l1 = peer_ref[:, 2:3]
        mm = jnp.maximum(m0, m1)
        lse = mm + jnp.log(s0 * jnp.exp(m0 - mm) + s1 * jnp.exp(m1 - mm))
        out_ref[...] = lse - (l0 + l1)

    out = pl.pallas_call(
        body,
        out_shape=jax.ShapeDtypeStruct((T, 1), jnp.float32),
        in_specs=[
            pl.BlockSpec(memory_space=pltpu.VMEM),
            pl.BlockSpec(memory_space=pltpu.VMEM),
            pl.BlockSpec(memory_space=pltpu.VMEM),
        ],
        out_specs=pl.BlockSpec(memory_space=pltpu.VMEM),
        scratch_shapes=[
            pltpu.VMEM((T, 3), jnp.float32),
            pltpu.VMEM((T, 3), jnp.float32),
            pltpu.SemaphoreType.DMA,
            pltpu.SemaphoreType.DMA,
        ],
        compiler_params=pltpu.CompilerParams(
            collective_id=0, vmem_limit_bytes=100 * 1024 * 1024
        ),
    )(x, W, labels2)
    return out.reshape(T)


# device time: 28990 ns/iter; 1.1913x vs baseline; 1.1913x over previous
import jax
import jax.numpy as jnp
from jax import lax
from jax.experimental import pallas as pl
from jax.experimental.pallas import tpu as pltpu

T = 512
D = 1024
V_LOCAL = 8192
BV = 1024
GRID = V_LOCAL // BV


def kernel(x, W, labels):
    labels2 = labels.reshape(T, 1)

    def body(x_ref, w_ref, lab_ref, out_ref, xb_ref, acc_ref, peer_ref, send_sem, recv_sem):
        i = pl.program_id(0)
        my_x = lax.axis_index("x")
        my_y = lax.axis_index("y")
        my_z = lax.axis_index("z")
        partner = (my_x, my_y, 1 - my_z)

        @pl.when(i == 0)
        def _():
            xb_ref[...] = x_ref[...].astype(jnp.bfloat16)
            barrier = pltpu.get_barrier_semaphore()
            pl.semaphore_signal(
                barrier, inc=1, device_id=partner, device_id_type=pl.DeviceIdType.MESH
            )

        wb = w_ref[...].astype(jnp.bfloat16)
        logits = jnp.dot(xb_ref[...], wb, preferred_element_type=jnp.float32)
        s_i = jnp.sum(jnp.exp(logits), axis=1, keepdims=True)
        ids = lax.broadcasted_iota(jnp.int32, (T, BV), 1) + (i * BV + my_z * V_LOCAL)
        lab_i = jnp.sum(
            jnp.where(ids == lab_ref[...], logits, 0.0), axis=1, keepdims=True
        )
        update = jnp.concatenate([s_i, lab_i], axis=1)

        @pl.when(i == 0)
        def _():
            acc_ref[...] = update

        @pl.when(i > 0)
        def _():
            acc_ref[...] += update

        @pl.when(i == GRID - 1)
        def _():
            barrier = pltpu.get_barrier_semaphore()
            pl.semaphore_wait(barrier, 1)
            rdma = pltpu.make_async_remote_copy(
                src_ref=acc_ref,
                dst_ref=peer_ref,
                send_sem=send_sem,
                recv_sem=recv_sem,
                device_id=partner,
                device_id_type=pl.DeviceIdType.MESH,
            )
            rdma.start()
            rdma.wait()
            s = acc_ref[:, 0:1] + peer_ref[:, 0:1]
            lab = acc_ref[:, 1:2] + peer_ref[:, 1:2]
            out_ref[...] = jnp.log(s) - lab

    out = pl.pallas_call(
        body,
        grid=(GRID,),
        out_shape=jax.ShapeDtypeStruct((T, 1), jnp.float32),
        in_specs=[
            pl.BlockSpec((T, D), lambda i: (0, 0)),
            pl.BlockSpec((D, BV), lambda i: (0, i)),
            pl.BlockSpec((T, 1), lambda i: (0, 0)),
        ],
        out_specs=pl.BlockSpec((T, 1), lambda i: (0, 0)),
        scratch_shapes=[
            pltpu.VMEM((T, D), jnp.bfloat16),
            pltpu.VMEM((T, 2), jnp.float32),
            pltpu.VMEM((T, 2), jnp.float32),
            pltpu.SemaphoreType.DMA,
            pltpu.SemaphoreType.DMA,
        ],
        compiler_params=pltpu.CompilerParams(
            collective_id=0, vmem_limit_bytes=100 * 1024 * 1024
        ),
    )(x, W, labels2)
    return out.reshape(T)
